# Initial kernel scaffold; baseline (speedup 1.0000x reference)
import jax
import jax.numpy as jnp
from jax.experimental import pallas as pl


def kernel(x, Wg, W1, W2):
    raise NotImplementedError("write your pallas kernel here")


# speedup vs baseline: 3.4911x; 3.4911x over previous
import functools

import jax
import jax.numpy as jnp
from jax.experimental import pallas as pl
from jax.experimental.pallas import tpu as pltpu

E = 8
K = 2
D = 768
L = 2048
H = 4 * D
HC = 4
HCD = H // HC


def _router_body(x_ref, wg_ref, w_ref, loss_ref):
    x = x_ref[...]
    wg = wg_ref[...]
    logits = jnp.dot(x, wg, preferred_element_type=jnp.float32)
    lane = jax.lax.broadcasted_iota(jnp.int32, (L, E), 1)
    m1 = jnp.max(logits, axis=1, keepdims=True)
    i1 = jnp.min(jnp.where(logits == m1, lane, E), axis=1, keepdims=True)
    oh1 = (lane == i1).astype(jnp.float32)
    logits2 = jnp.where(lane == i1, -jnp.inf, logits)
    m2 = jnp.max(logits2, axis=1, keepdims=True)
    i2 = jnp.min(jnp.where(logits2 == m2, lane, E), axis=1, keepdims=True)
    oh2 = (lane == i2).astype(jnp.float32)
    a = jnp.exp(m2 - m1)
    g1 = 1.0 / (1.0 + a)
    g2 = a / (1.0 + a)
    w_ref[...] = oh1 * g1 + oh2 * g2
    counts = jnp.sum(oh1 + oh2, axis=0, keepdims=True)
    cn = counts / (K * L)
    loss_ref[...] = jnp.sum((cn - 1.0 / E) ** 2, axis=1, keepdims=True) / E


def _gelu_exact(h):
    return 0.5 * h * (1.0 + jax.lax.erf(h * (2.0 ** -0.5)))


def _ffn_body(x_ref, w1_ref, w2_ref, w_ref, out_ref):
    e = pl.program_id(0)
    hc = pl.program_id(1)
    x = x_ref[...]
    h = jnp.dot(x, w1_ref[0], preferred_element_type=jnp.float32)
    h = _gelu_exact(h)
    lane = jax.lax.broadcasted_iota(jnp.int32, (1, E), 1)
    wcol = jnp.sum(w_ref[...] * (lane == e).astype(jnp.float32),
                   axis=1, keepdims=True)
    y = jnp.dot(h * wcol, w2_ref[0], preferred_element_type=jnp.float32)

    @pl.when(jnp.logical_and(e == 0, hc == 0))
    def _():
        out_ref[...] = y

    @pl.when(jnp.logical_not(jnp.logical_and(e == 0, hc == 0)))
    def _():
        out_ref[...] = out_ref[...] + y


def kernel(x, Wg, W1, W2):
    x2 = x.reshape(L, D)

    w, loss = pl.pallas_call(
        _router_body,
        out_shape=(
            jax.ShapeDtypeStruct((L, E), jnp.float32),
            jax.ShapeDtypeStruct((1, 1), jnp.float32),
        ),
    )(x2, Wg)

    out = pl.pallas_call(
        _ffn_body,
        grid=(E, HC),
        in_specs=[
            pl.BlockSpec((L, D), lambda e, hc: (0, 0)),
            pl.BlockSpec((1, D, HCD), lambda e, hc: (e, 0, hc)),
            pl.BlockSpec((1, HCD, D), lambda e, hc: (e, hc, 0)),
            pl.BlockSpec((L, E), lambda e, hc: (0, 0)),
        ],
        out_specs=pl.BlockSpec((L, D), lambda e, hc: (0, 0)),
        out_shape=jax.ShapeDtypeStruct((L, D), jnp.float32),
    )(x2, W1, W2, w)

    return out.reshape(1, L, D), loss.reshape(())
